# baseline (speedup 1.0000x reference)
import functools

import jax
import jax.numpy as jnp
from jax import lax
from jax.experimental import pallas as pl
from jax.experimental.pallas import tpu as pltpu
from jax.experimental.pallas import tpu_sc as plsc

MM = 10000
KK = 16
NC = 2
NS = 16
NW = NC * NS
GPW = (MM * KK) // NW
CH = 128
NCH = 40
TAIL = GPW - (NCH - 1) * CH

BB = 400
GRID = MM // BB


def _mm(a, b):
    return jnp.dot(a, b, preferred_element_type=jnp.float32)


def _elu(v):
    return jnp.where(v > 0, v, jnp.exp(jnp.minimum(v, 0.0)) - 1.0)


def _make_sc_gather(D):
    mesh = plsc.VectorSubcoreMesh(core_axis_name="c", subcore_axis_name="s")


    @functools.partial(
        pl.kernel,
        out_type=jax.ShapeDtypeStruct((MM * KK, D), jnp.float32),
        mesh=mesh,
        scratch_types=[
            pltpu.VMEM((NCH, CH), jnp.int32),
            pltpu.VMEM((CH, D), jnp.float32),
            pltpu.VMEM((CH, D), jnp.float32),
            pltpu.VMEM((CH, D), jnp.float32),
            pltpu.SemaphoreType.DMA,
            pltpu.SemaphoreType.DMA,
            pltpu.SemaphoreType.DMA,
            pltpu.SemaphoreType.DMA,
            pltpu.SemaphoreType.DMA,
            pltpu.SemaphoreType.DMA,
        ],
    )
    def gk(table_hbm, idx_hbm, out_hbm, idx_v,
           buf0, buf1, buf2, g0, g1, g2, s0, s1, s2):
        wid = lax.axis_index("s") * NC + lax.axis_index("c")
        base = wid * GPW
        pltpu.sync_copy(idx_hbm.at[wid], idx_v)
        bufs, gsems, ssems = (buf0, buf1, buf2), (g0, g1, g2), (s0, s1, s2)

        pltpu.async_copy(table_hbm.at[idx_v.at[0]], buf0, g0)
        pltpu.async_copy(table_hbm.at[idx_v.at[1]], buf1, g1)
        pltpu.async_copy(table_hbm.at[idx_v.at[2]], buf2, g2)

        def triple(p, carry):
            for b in range(3):
                c = 3 * p + b
                buf, gs, ss = bufs[b], gsems[b], ssems[b]
                pltpu.make_async_copy(table_hbm.at[idx_v.at[c]], buf, gs).wait()
                pltpu.async_copy(
                    buf, out_hbm.at[pl.ds(base + c * CH, CH)], ss).wait()

                @pl.when(c + 3 < NCH)
                def _():
                    pltpu.async_copy(table_hbm.at[idx_v.at[c + 3]], buf, gs)
            return carry

        lax.fori_loop(0, (NCH - 1) // 3, triple, 0)
        b = (NCH - 1) % 3
        pltpu.make_async_copy(
            table_hbm.at[idx_v.at[NCH - 1]], bufs[b], gsems[b]).wait()
        pltpu.async_copy(
            bufs[b].at[pl.ds(0, TAIL)],
            out_hbm.at[pl.ds(base + (NCH - 1) * CH, TAIL)], ssems[b]).wait()

    return gk


def _gather128(table, idx3):
    return _make_sc_gather(128)(table, idx3)


def _gather256(table, idx3):
    return _make_sc_gather(256)(table, idx3)


def _geo(pos, x8):
    pos3 = pos.reshape(BB, KK, 16)
    sub = (pos3 - x8[:, None, :]).reshape(BB * KK, 16)
    dist = jnp.abs(sub)
    return sub, dist


def _stage0_body(x_ref, pos_ref, a_ref, s_ref, d_ref, b0_ref, w1_ref, b1_ref,
                 w2_ref, b2_ref, wf_ref, f_ref, y_ref, p16_ref):
    pos = pos_ref[:, 0:16]
    p16_ref[...] = pos
    sub, dist = _geo(pos, x_ref[...])
    h = _mm(pos, a_ref[...]) + _mm(sub, s_ref[...]) + _mm(dist, d_ref[...])
    h = jnp.maximum(h + b0_ref[...], 0.0)
    h = jnp.maximum(_mm(h, w1_ref[...]) + b1_ref[...], 0.0)
    h = jnp.maximum(_mm(h, w2_ref[...]) + b2_ref[...], 0.0)
    f = _elu(jnp.sum(h.reshape(BB, KK, 256), axis=1))
    f_ref[...] = f
    y_ref[...] = _mm(f, wf_ref[...])


def _stage_body(has_next, yg_ref, pos_ref, x_ref, cur_ref, a_ref, s_ref,
                d_ref, b_ref, *rest):
    if has_next:
        wf_ref, f_ref, y_ref = rest
    else:
        wf_ref = None
        (f_ref,) = rest
    pos = pos_ref[...]
    sub, dist = _geo(pos, x_ref[...])
    g = _mm(pos, a_ref[...]) + _mm(sub, s_ref[...]) + _mm(dist, d_ref[...])
    t = jnp.maximum(yg_ref[...] + g + b_ref[...], 0.0)
    f = _elu(jnp.sum(t.reshape(BB, KK, 256), axis=1)) + cur_ref[...]
    f_ref[...] = f
    if has_next:
        y_ref[...] = _mm(f, wf_ref[...])


def _final_body(f1_ref, f2_ref, f3_ref, f4_ref, wa_ref, wb_ref, wc_ref,
                wd_ref, bo0_ref, w1_ref, bo1_ref, w2_ref, bo2_ref, o_ref):
    h = (_mm(f1_ref[...], wa_ref[...]) + _mm(f2_ref[...], wb_ref[...])
         + _mm(f3_ref[...], wc_ref[...]) + _mm(f4_ref[...], wd_ref[...]))
    h = jnp.maximum(h + bo0_ref[...], 0.0)
    h = jnp.maximum(_mm(h, w1_ref[...]) + bo1_ref[...], 0.0)
    o_ref[...] = _mm(h, w2_ref[...]) + bo2_ref[...]


def _node_spec(n):
    return pl.BlockSpec((BB, n), lambda b: (b, 0))


def _edge_spec(n):
    return pl.BlockSpec((BB * KK, n), lambda b: (b, 0))


def _full_spec(shape):
    return pl.BlockSpec(shape, lambda b: (0, 0))


def _f32(shape):
    return jax.ShapeDtypeStruct(shape, jnp.float32)


def kernel(x, knn_idx_list, W0_0, b0_0, W0_1, b0_1, W0_2, b0_2,
           W1_0, b1_0, W2_0, b2_0, W3_0, b3_0,
           Wo0, bo0, Wo1, bo1, Wo2, bo2):
    xf = x[0]
    xpad = jnp.concatenate(
        [xf, jnp.zeros((MM, 13), jnp.float32)], axis=1)
    idx2 = knn_idx_list[0].reshape(NW, GPW)
    idx3 = jnp.pad(idx2, ((0, 0), (0, NCH * CH - GPW))).reshape(NW, NCH, CH)

    def pad16(w3):
        return jnp.pad(w3.T, ((0, 13), (0, 0)))

    A0 = pad16(W0_0[:, 0:3] + W0_0[:, 3:6])
    S0 = pad16(W0_0[:, 6:9])
    D0 = pad16(W0_0[:, 9:12])
    geo_ops = []
    for W in (W1_0, W2_0, W3_0):
        geo_ops.append((pad16(W[:, 256:259]), pad16(W[:, 259:262]),
                        pad16(W[:, 262:265])))
    Wf = [W[:, :256].T for W in (W1_0, W2_0, W3_0)]
    row = lambda v: v.reshape(1, -1)

    xpad128 = jnp.concatenate(
        [xf, jnp.zeros((MM, 125), jnp.float32)], axis=1)
    posg128 = _gather128(xpad128, idx3)

    wspecs0 = [_full_spec(s) for s in
               ((16, 64), (16, 64), (16, 64), (1, 64), (64, 128), (1, 128),
                (128, 256), (1, 256), (256, 256))]
    f1, y1, posg = pl.pallas_call(
        _stage0_body,
        grid=(GRID,),
        in_specs=[_node_spec(16), _edge_spec(128)] + wspecs0,
        out_specs=[_node_spec(256), _node_spec(256), _edge_spec(16)],
        out_shape=[_f32((MM, 256)), _f32((MM, 256)), _f32((MM * KK, 16))],
    )(xpad, posg128, A0, S0, D0, row(b0_0), W0_1.T, row(b0_1),
      W0_2.T, row(b0_2), Wf[0])

    feats = [f1]
    cur, y = f1, y1
    for i in (1, 2, 3):
        yg = _gather256(y, idx3)
        has_next = i < 3
        a, s, d = geo_ops[i - 1]
        b = row((b1_0, b2_0, b3_0)[i - 1])
        wspecs = [_full_spec((16, 256))] * 3 + [_full_spec((1, 256))]
        outs = [_f32((MM, 256))]
        ospecs = [_node_spec(256)]
        ins = [yg, posg, xpad, cur, a, s, d, b]
        in_specs = [_edge_spec(256), _edge_spec(16), _node_spec(16),
                    _node_spec(256)] + wspecs
        if has_next:
            ins.append(Wf[i])
            in_specs.append(_full_spec((256, 256)))
            outs.append(_f32((MM, 256)))
            ospecs.append(_node_spec(256))
        res = pl.pallas_call(
            functools.partial(_stage_body, has_next),
            grid=(GRID,),
            in_specs=in_specs,
            out_specs=ospecs,
            out_shape=outs,
        )(*ins)
        if has_next:
            cur, y = res
        else:
            (cur,) = res
        feats.append(cur)

    out = pl.pallas_call(
        _final_body,
        grid=(GRID,),
        in_specs=[_node_spec(256)] * 4 + [
            _full_spec((256, 256)), _full_spec((256, 256)),
            _full_spec((256, 256)), _full_spec((256, 256)),
            _full_spec((1, 256)), _full_spec((256, 256)), _full_spec((1, 256)),
            _full_spec((256, 256)), _full_spec((1, 256))],
        out_specs=_node_spec(256),
        out_shape=_f32((MM, 256)),
    )(feats[0], feats[1], feats[2], feats[3],
      Wo0[:, 0:256].T, Wo0[:, 256:512].T, Wo0[:, 512:768].T, Wo0[:, 768:1024].T,
      row(bo0), Wo1.T, row(bo1), Wo2.T, row(bo2))
    return out[None]


# speedup vs baseline: 1.1808x; 1.1808x over previous
import functools

import jax
import jax.numpy as jnp
from jax import lax
from jax.experimental import pallas as pl
from jax.experimental.pallas import tpu as pltpu
from jax.experimental.pallas import tpu_sc as plsc

MM = 10000
KK = 16
NC = 2
NS = 16
NW = NC * NS
GPW = (MM * KK) // NW
CH = 128
NCH = 40
TAIL = GPW - (NCH - 1) * CH

BB = 400
GRID = MM // BB


def _mm(a, b):
    return jnp.dot(a, b, preferred_element_type=jnp.float32)


def _elu(v):
    return jnp.where(v > 0, v, jnp.exp(jnp.minimum(v, 0.0)) - 1.0)


def _make_sc_gather(row_shape, dtype):
    mesh = plsc.VectorSubcoreMesh(core_axis_name="c", subcore_axis_name="s")


    @functools.partial(
        pl.kernel,
        out_type=jax.ShapeDtypeStruct((MM * KK,) + row_shape, dtype),
        mesh=mesh,
        scratch_types=[
            pltpu.VMEM((NCH, CH), jnp.int32),
            pltpu.VMEM((CH,) + row_shape, dtype),
            pltpu.VMEM((CH,) + row_shape, dtype),
            pltpu.VMEM((CH,) + row_shape, dtype),
            pltpu.SemaphoreType.DMA,
            pltpu.SemaphoreType.DMA,
            pltpu.SemaphoreType.DMA,
            pltpu.SemaphoreType.DMA,
            pltpu.SemaphoreType.DMA,
            pltpu.SemaphoreType.DMA,
        ],
    )
    def gk(table_hbm, idx_hbm, out_hbm, idx_v,
           buf0, buf1, buf2, g0, g1, g2, s0, s1, s2):
        wid = lax.axis_index("s") * NC + lax.axis_index("c")
        base = wid * GPW
        pltpu.sync_copy(idx_hbm.at[wid], idx_v)
        bufs, gsems, ssems = (buf0, buf1, buf2), (g0, g1, g2), (s0, s1, s2)

        pltpu.async_copy(table_hbm.at[idx_v.at[0]], buf0, g0)
        pltpu.async_copy(table_hbm.at[idx_v.at[1]], buf1, g1)
        pltpu.async_copy(table_hbm.at[idx_v.at[2]], buf2, g2)

        def triple(p, carry):
            for b in range(3):
                c = 3 * p + b
                buf, gs, ss = bufs[b], gsems[b], ssems[b]
                pltpu.make_async_copy(table_hbm.at[idx_v.at[c]], buf, gs).wait()
                pltpu.async_copy(
                    buf, out_hbm.at[pl.ds(base + c * CH, CH)], ss).wait()

                @pl.when(c + 3 < NCH)
                def _():
                    pltpu.async_copy(table_hbm.at[idx_v.at[c + 3]], buf, gs)
            return carry

        lax.fori_loop(0, (NCH - 1) // 3, triple, 0)
        b = (NCH - 1) % 3
        pltpu.make_async_copy(
            table_hbm.at[idx_v.at[NCH - 1]], bufs[b], gsems[b]).wait()
        pltpu.async_copy(
            bufs[b].at[pl.ds(0, TAIL)],
            out_hbm.at[pl.ds(base + (NCH - 1) * CH, TAIL)], ssems[b]).wait()

    return gk


def _gather128(table, idx3):
    return _make_sc_gather((128,), jnp.float32)(table, idx3)


def _gatherpk(table, idx3):
    return _make_sc_gather((128,), jnp.int32)(table, idx3)


def _pack_bf16(y):
    def q(v):
        u = jax.lax.bitcast_convert_type(v, jnp.int32)
        r = jax.lax.shift_right_logical(u, 16) & 1
        return jax.lax.shift_right_logical(u + 0x7FFF + r, 16)

    return q(y[:, :128]) | jax.lax.shift_left(q(y[:, 128:]), 16)


def _unpack_bf16(w):
    lo = jax.lax.bitcast_convert_type(jax.lax.shift_left(w, 16), jnp.float32)
    hi = jax.lax.bitcast_convert_type(w & jnp.int32(-65536), jnp.float32)
    return lo, hi


def _geo(pos, x8):
    pos3 = pos.reshape(BB, KK, 16)
    sub = (pos3 - x8[:, None, :]).reshape(BB * KK, 16)
    dist = jnp.abs(sub)
    return sub, dist


def _stage0_body(x_ref, pos_ref, a_ref, s_ref, d_ref, b0_ref, w1_ref, b1_ref,
                 w2_ref, b2_ref, wf_ref, f_ref, y_ref, p16_ref):
    pos = pos_ref[:, 0:16]
    p16_ref[...] = pos
    sub, dist = _geo(pos, x_ref[...])
    h = _mm(pos, a_ref[...]) + _mm(sub, s_ref[...]) + _mm(dist, d_ref[...])
    h = jnp.maximum(h + b0_ref[...], 0.0)
    h = jnp.maximum(_mm(h, w1_ref[...]) + b1_ref[...], 0.0)
    h = jnp.maximum(_mm(h, w2_ref[...]) + b2_ref[...], 0.0)
    f = _elu(jnp.sum(h.reshape(BB, KK, 256), axis=1))
    f_ref[...] = f
    y_ref[...] = _pack_bf16(_mm(f, wf_ref[...]))


def _stage_body(has_next, yg_ref, pos_ref, x_ref, cur_ref, a_ref, s_ref,
                d_ref, b_ref, *rest):
    if has_next:
        wf_ref, f_ref, y_ref = rest
    else:
        wf_ref = None
        (f_ref,) = rest
    pos = pos_ref[...]
    sub, dist = _geo(pos, x_ref[...])
    g = _mm(pos, a_ref[...]) + _mm(sub, s_ref[...]) + _mm(dist, d_ref[...])
    g = g + b_ref[...]
    ylo, yhi = _unpack_bf16(yg_ref[...])
    tlo = jnp.maximum(ylo + g[:, :128], 0.0)
    thi = jnp.maximum(yhi + g[:, 128:], 0.0)
    s = jnp.concatenate(
        [jnp.sum(tlo.reshape(BB, KK, 128), axis=1),
         jnp.sum(thi.reshape(BB, KK, 128), axis=1)], axis=1)
    f = _elu(s) + cur_ref[...]
    f_ref[...] = f
    if has_next:
        y_ref[...] = _pack_bf16(_mm(f, wf_ref[...]))


def _final_body(f1_ref, f2_ref, f3_ref, f4_ref, wa_ref, wb_ref, wc_ref,
                wd_ref, bo0_ref, w1_ref, bo1_ref, w2_ref, bo2_ref, o_ref):
    h = (_mm(f1_ref[...], wa_ref[...]) + _mm(f2_ref[...], wb_ref[...])
         + _mm(f3_ref[...], wc_ref[...]) + _mm(f4_ref[...], wd_ref[...]))
    h = jnp.maximum(h + bo0_ref[...], 0.0)
    h = jnp.maximum(_mm(h, w1_ref[...]) + bo1_ref[...], 0.0)
    o_ref[...] = _mm(h, w2_ref[...]) + bo2_ref[...]


def _node_spec(n):
    return pl.BlockSpec((BB, n), lambda b: (b, 0))


def _edge_spec(n):
    return pl.BlockSpec((BB * KK, n), lambda b: (b, 0))


def _full_spec(shape):
    return pl.BlockSpec(shape, lambda b: (0, 0))


def _f32(shape):
    return jax.ShapeDtypeStruct(shape, jnp.float32)


def kernel(x, knn_idx_list, W0_0, b0_0, W0_1, b0_1, W0_2, b0_2,
           W1_0, b1_0, W2_0, b2_0, W3_0, b3_0,
           Wo0, bo0, Wo1, bo1, Wo2, bo2):
    xf = x[0]
    xpad = jnp.concatenate(
        [xf, jnp.zeros((MM, 13), jnp.float32)], axis=1)
    idx2 = knn_idx_list[0].reshape(NW, GPW)
    idx3 = jnp.pad(idx2, ((0, 0), (0, NCH * CH - GPW))).reshape(NW, NCH, CH)

    def pad16(w3):
        return jnp.pad(w3.T, ((0, 13), (0, 0)))

    A0 = pad16(W0_0[:, 0:3] + W0_0[:, 3:6])
    S0 = pad16(W0_0[:, 6:9])
    D0 = pad16(W0_0[:, 9:12])
    geo_ops = []
    for W in (W1_0, W2_0, W3_0):
        geo_ops.append((pad16(W[:, 256:259]), pad16(W[:, 259:262]),
                        pad16(W[:, 262:265])))
    Wf = [W[:, :256].T for W in (W1_0, W2_0, W3_0)]
    row = lambda v: v.reshape(1, -1)

    xpad128 = jnp.concatenate(
        [xf, jnp.zeros((MM, 125), jnp.float32)], axis=1)
    posg128 = _gather128(xpad128, idx3)

    wspecs0 = [_full_spec(s) for s in
               ((16, 64), (16, 64), (16, 64), (1, 64), (64, 128), (1, 128),
                (128, 256), (1, 256), (256, 256))]
    _ybf = jax.ShapeDtypeStruct((MM, 128), jnp.int32)
    f1, y1, posg = pl.pallas_call(
        _stage0_body,
        grid=(GRID,),
        in_specs=[_node_spec(16), _edge_spec(128)] + wspecs0,
        out_specs=[_node_spec(256), _node_spec(128), _edge_spec(16)],
        out_shape=[_f32((MM, 256)), _ybf, _f32((MM * KK, 16))],
    )(xpad, posg128, A0, S0, D0, row(b0_0), W0_1.T, row(b0_1),
      W0_2.T, row(b0_2), Wf[0])

    feats = [f1]
    cur, y = f1, y1
    for i in (1, 2, 3):
        yg = _gatherpk(y, idx3)
        has_next = i < 3
        a, s, d = geo_ops[i - 1]
        b = row((b1_0, b2_0, b3_0)[i - 1])
        wspecs = [_full_spec((16, 256))] * 3 + [_full_spec((1, 256))]
        outs = [_f32((MM, 256))]
        ospecs = [_node_spec(256)]
        ins = [yg, posg, xpad, cur, a, s, d, b]
        in_specs = [_edge_spec(128), _edge_spec(16), _node_spec(16),
                    _node_spec(256)] + wspecs
        if has_next:
            ins.append(Wf[i])
            in_specs.append(_full_spec((256, 256)))
            outs.append(_ybf)
            ospecs.append(_node_spec(128))
        res = pl.pallas_call(
            functools.partial(_stage_body, has_next),
            grid=(GRID,),
            in_specs=in_specs,
            out_specs=ospecs,
            out_shape=outs,
        )(*ins)
        if has_next:
            cur, y = res
        else:
            (cur,) = res
        feats.append(cur)

    out = pl.pallas_call(
        _final_body,
        grid=(GRID,),
        in_specs=[_node_spec(256)] * 4 + [
            _full_spec((256, 256)), _full_spec((256, 256)),
            _full_spec((256, 256)), _full_spec((256, 256)),
            _full_spec((1, 256)), _full_spec((256, 256)), _full_spec((1, 256)),
            _full_spec((256, 256)), _full_spec((1, 256))],
        out_specs=_node_spec(256),
        out_shape=_f32((MM, 256)),
    )(feats[0], feats[1], feats[2], feats[3],
      Wo0[:, 0:256].T, Wo0[:, 256:512].T, Wo0[:, 512:768].T, Wo0[:, 768:1024].T,
      row(bo0), Wo1.T, row(bo1), Wo2.T, row(bo2))
    return out[None]


# speedup vs baseline: 1.1917x; 1.0093x over previous
import functools

import jax
import jax.numpy as jnp
from jax import lax
from jax.experimental import pallas as pl
from jax.experimental.pallas import tpu as pltpu
from jax.experimental.pallas import tpu_sc as plsc

MM = 10000
KK = 16
NC = 2
NS = 16
NW = NC * NS
CH = 128

BB = 400

HN = (4800, 5200)
HE = (HN[0] * KK, HN[1] * KK)
HPW = (HE[0] // NW, HE[1] // NW)
HNCH = (19, 21)
HTAIL = (HPW[0] - (HNCH[0] - 1) * CH, HPW[1] - (HNCH[1] - 1) * CH)
HBLK = (HN[0] // BB, HN[1] // BB)


def _mm(a, b):
    return jnp.dot(a, b, preferred_element_type=jnp.float32)


def _elu(v):
    return jnp.where(v > 0, v, jnp.exp(jnp.minimum(v, 0.0)) - 1.0)


def _make_sc_gather(row_shape, dtype, npw, nch, tail, nedges):
    mesh = plsc.VectorSubcoreMesh(core_axis_name="c", subcore_axis_name="s")

    @functools.partial(
        pl.kernel,
        out_type=jax.ShapeDtypeStruct((nedges,) + row_shape, dtype),
        mesh=mesh,
        scratch_types=[
            pltpu.VMEM((nch, CH), jnp.int32),
            pltpu.VMEM((CH,) + row_shape, dtype),
            pltpu.VMEM((CH,) + row_shape, dtype),
            pltpu.VMEM((CH,) + row_shape, dtype),
            pltpu.SemaphoreType.DMA,
            pltpu.SemaphoreType.DMA,
            pltpu.SemaphoreType.DMA,
            pltpu.SemaphoreType.DMA,
            pltpu.SemaphoreType.DMA,
            pltpu.SemaphoreType.DMA,
        ],
    )
    def gk(table_hbm, idx_hbm, out_hbm, idx_v,
           buf0, buf1, buf2, g0, g1, g2, s0, s1, s2):
        wid = lax.axis_index("s") * NC + lax.axis_index("c")
        base = wid * npw
        pltpu.sync_copy(idx_hbm.at[wid], idx_v)
        bufs, gsems, ssems = (buf0, buf1, buf2), (g0, g1, g2), (s0, s1, s2)

        pltpu.async_copy(table_hbm.at[idx_v.at[0]], buf0, g0)
        pltpu.async_copy(table_hbm.at[idx_v.at[1]], buf1, g1)
        pltpu.async_copy(table_hbm.at[idx_v.at[2]], buf2, g2)

        def process(c, b, dyn_next):
            buf, gs, ss = bufs[b], gsems[b], ssems[b]
            pltpu.make_async_copy(table_hbm.at[idx_v.at[c]], buf, gs).wait()
            pltpu.async_copy(
                buf, out_hbm.at[pl.ds(base + c * CH, CH)], ss).wait()
            if dyn_next:
                @pl.when(c + 3 < nch)
                def _():
                    pltpu.async_copy(table_hbm.at[idx_v.at[c + 3]], buf, gs)
            elif c + 3 < nch:
                pltpu.async_copy(table_hbm.at[idx_v.at[c + 3]], buf, gs)

        n_triples = (nch - 1) // 3

        def triple(p, carry):
            for b in range(3):
                process(3 * p + b, b, True)
            return carry

        lax.fori_loop(0, n_triples, triple, 0)
        for c in range(3 * n_triples, nch - 1):
            process(c, c % 3, False)
        b = (nch - 1) % 3
        pltpu.make_async_copy(
            table_hbm.at[idx_v.at[nch - 1]], bufs[b], gsems[b]).wait()
        pltpu.async_copy(
            bufs[b].at[pl.ds(0, tail)],
            out_hbm.at[pl.ds(base + (nch - 1) * CH, tail)], ssems[b]).wait()

    return gk


def _gather128(table, idx3, h):
    return _make_sc_gather((128,), jnp.float32,
                           HPW[h], HNCH[h], HTAIL[h], HE[h])(table, idx3)


def _gatherpk(table, idx3, h):
    return _make_sc_gather((128,), jnp.int32,
                           HPW[h], HNCH[h], HTAIL[h], HE[h])(table, idx3)


def _pack_bf16(y):
    def q(v):
        u = jax.lax.bitcast_convert_type(v, jnp.int32)
        r = jax.lax.shift_right_logical(u, 16) & 1
        return jax.lax.shift_right_logical(u + 0x7FFF + r, 16)

    return q(y[:, :128]) | jax.lax.shift_left(q(y[:, 128:]), 16)


def _unpack_bf16(w):
    lo = jax.lax.bitcast_convert_type(jax.lax.shift_left(w, 16), jnp.float32)
    hi = jax.lax.bitcast_convert_type(w & jnp.int32(-65536), jnp.float32)
    return lo, hi


def _geo(pos, x8):
    pos3 = pos.reshape(BB, KK, 16)
    sub = (pos3 - x8[:, None, :]).reshape(BB * KK, 16)
    dist = jnp.abs(sub)
    return sub, dist


def _stage0_body(x_ref, pos_ref, a_ref, s_ref, d_ref, b0_ref, w1_ref, b1_ref,
                 w2_ref, b2_ref, wf_ref, f_ref, y_ref, p16_ref):
    pos = pos_ref[:, 0:16]
    p16_ref[...] = pos
    sub, dist = _geo(pos, x_ref[...])
    h = _mm(pos, a_ref[...]) + _mm(sub, s_ref[...]) + _mm(dist, d_ref[...])
    h = jnp.maximum(h + b0_ref[...], 0.0)
    h = jnp.maximum(_mm(h, w1_ref[...]) + b1_ref[...], 0.0)
    h = jnp.maximum(_mm(h, w2_ref[...]) + b2_ref[...], 0.0)
    f = _elu(jnp.sum(h.reshape(BB, KK, 256), axis=1))
    f_ref[...] = f
    y_ref[...] = _pack_bf16(_mm(f, wf_ref[...]))


def _stage_body(has_next, yg_ref, pos_ref, x_ref, cur_ref, a_ref, s_ref,
                d_ref, b_ref, *rest):
    if has_next:
        wf_ref, f_ref, y_ref = rest
    else:
        wf_ref = None
        (f_ref,) = rest
    pos = pos_ref[...]
    sub, dist = _geo(pos, x_ref[...])
    g = _mm(pos, a_ref[...]) + _mm(sub, s_ref[...]) + _mm(dist, d_ref[...])
    g = g + b_ref[...]
    ylo, yhi = _unpack_bf16(yg_ref[...])
    tlo = jnp.maximum(ylo + g[:, :128], 0.0)
    thi = jnp.maximum(yhi + g[:, 128:], 0.0)
    s = jnp.concatenate(
        [jnp.sum(tlo.reshape(BB, KK, 128), axis=1),
         jnp.sum(thi.reshape(BB, KK, 128), axis=1)], axis=1)
    f = _elu(s) + cur_ref[...]
    f_ref[...] = f
    if has_next:
        y_ref[...] = _pack_bf16(_mm(f, wf_ref[...]))


def _final_body(f1_ref, f2_ref, f3_ref, f4_ref, wa_ref, wb_ref, wc_ref,
                wd_ref, bo0_ref, w1_ref, bo1_ref, w2_ref, bo2_ref, o_ref):
    h = (_mm(f1_ref[...], wa_ref[...]) + _mm(f2_ref[...], wb_ref[...])
         + _mm(f3_ref[...], wc_ref[...]) + _mm(f4_ref[...], wd_ref[...]))
    h = jnp.maximum(h + bo0_ref[...], 0.0)
    h = jnp.maximum(_mm(h, w1_ref[...]) + bo1_ref[...], 0.0)
    o_ref[...] = _mm(h, w2_ref[...]) + bo2_ref[...]


def _node_spec(n, off=0):
    return pl.BlockSpec((BB, n), lambda b, off=off: (b + off, 0))


def _edge_spec(n):
    return pl.BlockSpec((BB * KK, n), lambda b: (b, 0))


def _full_spec(shape):
    return pl.BlockSpec(shape, lambda b: (0, 0))


def _f32(shape):
    return jax.ShapeDtypeStruct(shape, jnp.float32)


def kernel(x, knn_idx_list, W0_0, b0_0, W0_1, b0_1, W0_2, b0_2,
           W1_0, b1_0, W2_0, b2_0, W3_0, b3_0,
           Wo0, bo0, Wo1, bo1, Wo2, bo2):
    xf = x[0]
    xpad = jnp.concatenate(
        [xf, jnp.zeros((MM, 13), jnp.float32)], axis=1)
    idxflat = knn_idx_list[0].reshape(MM * KK)
    idx_h = []
    off = 0
    for h in range(2):
        i2 = idxflat[off:off + HE[h]].reshape(NW, HPW[h])
        i2 = jnp.pad(i2, ((0, 0), (0, HNCH[h] * CH - HPW[h])))
        idx_h.append(i2.reshape(NW, HNCH[h], CH))
        off += HE[h]

    def pad16(w3):
        return jnp.pad(w3.T, ((0, 13), (0, 0)))

    A0 = pad16(W0_0[:, 0:3] + W0_0[:, 3:6])
    S0 = pad16(W0_0[:, 6:9])
    D0 = pad16(W0_0[:, 9:12])
    geo_ops = []
    for W in (W1_0, W2_0, W3_0):
        geo_ops.append((pad16(W[:, 256:259]), pad16(W[:, 259:262]),
                        pad16(W[:, 262:265])))
    Wf = [W[:, :256].T for W in (W1_0, W2_0, W3_0)]
    row = lambda v: v.reshape(1, -1)

    xpad128 = jnp.concatenate(
        [xf, jnp.zeros((MM, 125), jnp.float32)], axis=1)
    posg128 = [_gather128(xpad128, idx_h[h], h) for h in range(2)]

    wspecs0 = [_full_spec(s) for s in
               ((16, 64), (16, 64), (16, 64), (1, 64), (64, 128), (1, 128),
                (128, 256), (1, 256), (256, 256))]
    noff = (0, HBLK[0])
    f1s, y1s, posg = [], [], []
    for h in range(2):
        fh, yh, ph = pl.pallas_call(
            _stage0_body,
            grid=(HBLK[h],),
            in_specs=[_node_spec(16, noff[h]), _edge_spec(128)] + wspecs0,
            out_specs=[_node_spec(256), _node_spec(128), _edge_spec(16)],
            out_shape=[_f32((HN[h], 256)),
                       jax.ShapeDtypeStruct((HN[h], 128), jnp.int32),
                       _f32((HE[h], 16))],
        )(xpad, posg128[h], A0, S0, D0, row(b0_0), W0_1.T, row(b0_1),
          W0_2.T, row(b0_2), Wf[0])
        f1s.append(fh)
        y1s.append(yh)
        posg.append(ph)

    feats = [f1s]
    cur = f1s
    y = jnp.concatenate(y1s, axis=0)
    for i in (1, 2, 3):
        has_next = i < 3
        a, s, d = geo_ops[i - 1]
        b = row((b1_0, b2_0, b3_0)[i - 1])
        wspecs = [_full_spec((16, 256))] * 3 + [_full_spec((1, 256))]
        new_cur, new_y = [], []
        for h in range(2):
            yg = _gatherpk(y, idx_h[h], h)
            ins = [yg, posg[h], xpad, cur[h], a, s, d, b]
            in_specs = [_edge_spec(128), _edge_spec(16),
                        _node_spec(16, noff[h]), _node_spec(256)] + wspecs
            outs = [_f32((HN[h], 256))]
            ospecs = [_node_spec(256)]
            if has_next:
                ins.append(Wf[i])
                in_specs.append(_full_spec((256, 256)))
                outs.append(jax.ShapeDtypeStruct((HN[h], 128), jnp.int32))
                ospecs.append(_node_spec(128))
            res = pl.pallas_call(
                functools.partial(_stage_body, has_next),
                grid=(HBLK[h],),
                in_specs=in_specs,
                out_specs=ospecs,
                out_shape=outs,
            )(*ins)
            if has_next:
                new_cur.append(res[0])
                new_y.append(res[1])
            else:
                new_cur.append(res[0])
        cur = new_cur
        feats.append(cur)
        if has_next:
            y = jnp.concatenate(new_y, axis=0)

    outs = []
    for h in range(2):
        oh = pl.pallas_call(
            _final_body,
            grid=(HBLK[h],),
            in_specs=[_node_spec(256)] * 4 + [
                _full_spec((256, 256)), _full_spec((256, 256)),
                _full_spec((256, 256)), _full_spec((256, 256)),
                _full_spec((1, 256)), _full_spec((256, 256)),
                _full_spec((1, 256)), _full_spec((256, 256)),
                _full_spec((1, 256))],
            out_specs=_node_spec(256),
            out_shape=_f32((HN[h], 256)),
        )(feats[0][h], feats[1][h], feats[2][h], feats[3][h],
          Wo0[:, 0:256].T, Wo0[:, 256:512].T, Wo0[:, 512:768].T,
          Wo0[:, 768:1024].T, row(bo0), Wo1.T, row(bo1), Wo2.T, row(bo2))
        outs.append(oh)
    return jnp.concatenate(outs, axis=0)[None]
